# baseline (speedup 1.0000x reference)
import jax
import jax.numpy as jnp
from jax import lax
from jax.experimental import pallas as pl
from jax.experimental.pallas import tpu as pltpu
from jax.experimental.pallas import tpu_sc as plsc

SEQ = 200
DIM = 32
BATCH = 4096
VOCAB = 1000000
NW = 32
B_BLK = BATCH // NW
NEB = DIM // 8
SCALE = float(DIM) ** 0.5


NBUF = 4
NGRP = SEQ // NBUF


def _body(idx_hbm, table_hbm, pos_hbm, out_hbm, idxr_v, idxt_v, pos_v, rows_v, out_t, gsem, osem):
    cidx = lax.axis_index("c")
    sidx = lax.axis_index("s")
    w = sidx * 2 + cidx
    b0 = pl.multiple_of(w * B_BLK, 8)

    pltpu.sync_copy(idx_hbm.at[pl.ds(b0, B_BLK)], idxr_v)
    pltpu.sync_copy(pos_hbm, pos_v)

    iota = jnp.arange(16, dtype=jnp.int32)
    eb_lo = iota // 8
    e8_lo = iota % 8
    eb_hi = (iota + 16) // 8
    e8_hi = (iota + 16) % 8

    def tr_body(s, carry):
        svec = jnp.full((16,), 0, jnp.int32) + s
        for g in range(B_BLK // 16):
            v = plsc.load_gather(idxr_v, [iota + 16 * g, svec])
            idxt_v[s, pl.ds(16 * g, 16)] = v
        return carry

    lax.fori_loop(0, SEQ, tr_body, 0)

    def gather_start(s, slot):
        pltpu.async_copy(table_hbm.at[idxt_v.at[s]], rows_v.at[slot], gsem.at[slot])

    def gather_drain(slot):
        pltpu.make_async_copy(
            table_hbm.at[pl.ds(0, B_BLK)], rows_v.at[slot], gsem.at[slot]
        ).wait()

    def out_group_start(g, par):
        pltpu.async_copy(
            out_t.at[par, :, :, :, pl.ds(0, 128)],
            out_hbm.at[pl.ds(NBUF * g, NBUF), :, w],
            osem.at[par],
        )

    def out_group_drain(par):
        pltpu.make_async_copy(
            out_t.at[par, :, :, :, pl.ds(0, 128)],
            out_hbm.at[pl.ds(0, NBUF), :, 0],
            osem.at[par],
        ).wait()

    for b in range(NBUF):
        gather_start(b, b)

    def group(g, carry):
        par = lax.rem(g, 2)
        @pl.when(g >= 2)
        def _():
            out_group_drain(par)

        for b in range(NBUF):
            s = NBUF * g + b
            gather_drain(b)
            p0 = pos_v[s, pl.ds(0, 16)]
            p1 = pos_v[s, pl.ds(16, 16)]
            tile = out_t.at[par, b]
            @plsc.parallel_loop(0, B_BLK, step=1, unroll=8)
            def _(bl):
                blv = jnp.full((16,), 0, jnp.int32) + bl
                v0 = rows_v[b, bl, pl.ds(0, 16)]
                v1 = rows_v[b, bl, pl.ds(16, 16)]
                plsc.store_scatter(tile, [eb_lo, e8_lo, blv], v0 * SCALE + p0)
                plsc.store_scatter(tile, [eb_hi, e8_hi, blv], v1 * SCALE + p1)
            @pl.when(g < NGRP - 1)
            def _():
                gather_start(s + NBUF, b)

        out_group_start(g, par)
        return carry

    lax.fori_loop(0, NGRP, group, 0)
    out_group_drain(0)
    out_group_drain(1)


A_CHUNK = 160
A_OUT = A_CHUNK // 4
A_NCH = VOCAB // A_CHUNK
A_ITER = (A_NCH + NW - 1) // NW


def _repack_body(table_hbm, out_hbm, src_v, dst_v):
    cidx = lax.axis_index("c")
    sidx = lax.axis_index("s")
    w = sidx * 2 + cidx

    def chunk(i, carry):
        c = w + NW * i

        @pl.when(c < A_NCH)
        def _():
            r0 = pl.multiple_of(c * A_CHUNK, 8)
            o0 = pl.multiple_of(c * A_OUT, 8)
            pltpu.sync_copy(table_hbm.at[pl.ds(r0, A_CHUNK)], src_v)

            @plsc.parallel_loop(0, A_CHUNK, step=1, unroll=8)
            def _(r):
                dst_v[r // 4, pl.ds((r % 4) * 32, 16)] = src_v[r, pl.ds(0, 16)]
                dst_v[r // 4, pl.ds((r % 4) * 32 + 16, 16)] = src_v[r, pl.ds(16, 16)]

            pltpu.sync_copy(dst_v, out_hbm.at[pl.ds(o0, A_OUT)])

        return carry

    lax.fori_loop(0, A_ITER, chunk, 0)


@jax.jit
def kernel(inputs, token_table, pos_table):
    mesh = plsc.VectorSubcoreMesh(core_axis_name="c", subcore_axis_name="s")
    packed = pl.kernel(
        _repack_body,
        out_type=jax.ShapeDtypeStruct((VOCAB * DIM // 128, 128), jnp.float32),
        mesh=mesh,
        scratch_types=[
            pltpu.VMEM((A_CHUNK, DIM), jnp.float32),
            pltpu.VMEM((A_OUT, 128), jnp.float32),
        ],
        compiler_params=pltpu.CompilerParams(use_tc_tiling_on_sc=True),
    )(token_table)
    tt = packed.reshape(VOCAB, DIM)
    out5 = pl.kernel(
        _body,
        out_type=jax.ShapeDtypeStruct((SEQ, NEB, BATCH // 128, 8, 128), jnp.float32),
        mesh=mesh,
        scratch_types=[
            pltpu.VMEM((B_BLK, SEQ), jnp.int32),
            pltpu.VMEM((SEQ, B_BLK), jnp.int32),
            pltpu.VMEM((SEQ, DIM), jnp.float32),
            pltpu.VMEM((NBUF, B_BLK, DIM), jnp.float32),
            pltpu.VMEM((2, NBUF, NEB, 8, 129), jnp.float32),
            pltpu.SemaphoreType.DMA((NBUF,)),
            pltpu.SemaphoreType.DMA((2,)),
        ],
        compiler_params=pltpu.CompilerParams(
            use_tc_tiling_on_sc=False, needs_layout_passes=False
        ),
    )(inputs, tt, pos_table)
    return out5.transpose(2, 4, 0, 1, 3).reshape(BATCH, SEQ, DIM)


# speedup vs baseline: 1.0240x; 1.0240x over previous
import jax
import jax.numpy as jnp
from jax import lax
from jax.experimental import pallas as pl
from jax.experimental.layout import Layout, with_layout_constraint
from jax.experimental.pallas import tpu as pltpu
from jax.experimental.pallas import tpu_sc as plsc

SEQ = 200
DIM = 32
BATCH = 4096
VOCAB = 1000000
NW = 32
B_BLK = BATCH // NW
NEB = DIM // 8
SCALE = float(DIM) ** 0.5


NBUF = 4
NGRP = SEQ // NBUF


def _body(idx_hbm, table_hbm, pos_hbm, out_hbm, idxr_v, idxt_v, pos_v, rows_v, out_t, gsem, osem):
    cidx = lax.axis_index("c")
    sidx = lax.axis_index("s")
    w = sidx * 2 + cidx
    b0 = pl.multiple_of(w * B_BLK, 8)

    pltpu.sync_copy(idx_hbm.at[pl.ds(b0, B_BLK)], idxr_v)
    pltpu.sync_copy(pos_hbm, pos_v)

    iota = jnp.arange(16, dtype=jnp.int32)
    eb_lo = iota // 8
    e8_lo = iota % 8
    eb_hi = (iota + 16) // 8
    e8_hi = (iota + 16) % 8

    def tr_body(s, carry):
        svec = jnp.full((16,), 0, jnp.int32) + s
        for g in range(B_BLK // 16):
            v = plsc.load_gather(idxr_v, [iota + 16 * g, svec])
            idxt_v[s, pl.ds(16 * g, 16)] = v
        return carry

    lax.fori_loop(0, SEQ, tr_body, 0)

    def gather_start(s, slot):
        pltpu.async_copy(table_hbm.at[idxt_v.at[s]], rows_v.at[slot], gsem.at[slot])

    def gather_drain(slot):
        pltpu.make_async_copy(
            table_hbm.at[pl.ds(0, B_BLK)], rows_v.at[slot], gsem.at[slot]
        ).wait()

    def out_group_start(g, par):
        pltpu.async_copy(
            out_t.at[par, :, :, :, pl.ds(0, 128)],
            out_hbm.at[pl.ds(NBUF * g, NBUF), :, w],
            osem.at[par],
        )

    def out_group_drain(par):
        pltpu.make_async_copy(
            out_t.at[par, :, :, :, pl.ds(0, 128)],
            out_hbm.at[pl.ds(0, NBUF), :, 0],
            osem.at[par],
        ).wait()

    def group(g, carry):
        par = lax.rem(g, 2)

        for b in range(NBUF):
            s = NBUF * g + b
            gather_start(s, b)
            gather_drain(b)
            p0 = pos_v[s, pl.ds(0, 16)]
            p1 = pos_v[s, pl.ds(16, 16)]
            tile = out_t.at[par, b]
            @plsc.parallel_loop(0, B_BLK, step=1, unroll=8)
            def _(bl):
                blv = jnp.full((16,), 0, jnp.int32) + bl
                v0 = rows_v[b, bl, pl.ds(0, 16)]
                v1 = rows_v[b, bl, pl.ds(16, 16)]
                plsc.store_scatter(tile, [eb_lo, e8_lo, blv], v0 * SCALE + p0)
                plsc.store_scatter(tile, [eb_hi, e8_hi, blv], v1 * SCALE + p1)

        out_group_start(g, par)
        out_group_drain(par)
        return carry

    lax.fori_loop(0, NGRP, group, 0)


@jax.jit
def kernel(inputs, token_table, pos_table):
    tt = token_table
    mesh = plsc.VectorSubcoreMesh(core_axis_name="c", subcore_axis_name="s")
    out5 = pl.kernel(
        _body,
        out_type=jax.ShapeDtypeStruct((SEQ, NEB, BATCH // 128, 8, 128), jnp.float32),
        mesh=mesh,
        scratch_types=[
            pltpu.VMEM((B_BLK, SEQ), jnp.int32),
            pltpu.VMEM((SEQ, B_BLK), jnp.int32),
            pltpu.VMEM((SEQ, DIM), jnp.float32),
            pltpu.VMEM((NBUF, B_BLK, DIM), jnp.float32),
            pltpu.VMEM((2, NBUF, NEB, 8, 129), jnp.float32),
            pltpu.SemaphoreType.DMA((NBUF,)),
            pltpu.SemaphoreType.DMA((2,)),
        ],
        compiler_params=pltpu.CompilerParams(
            use_tc_tiling_on_sc=False, needs_layout_passes=False
        ),
    )(inputs, tt, pos_table)
    return out5.transpose(2, 4, 0, 1, 3).reshape(BATCH, SEQ, DIM)


# speedup vs baseline: 1.3012x; 1.2707x over previous
import jax
import jax.numpy as jnp
from jax import lax
from jax.experimental import pallas as pl
from jax.experimental.layout import Layout, with_layout_constraint
from jax.experimental.pallas import tpu as pltpu
from jax.experimental.pallas import tpu_sc as plsc

SEQ = 200
DIM = 32
BATCH = 4096
VOCAB = 1000000
NW = 32
B_BLK = BATCH // NW
NEB = DIM // 8
SCALE = float(DIM) ** 0.5


NBUF = 4
NGRP = SEQ // NBUF


def _body(idx_hbm, table_hbm, pos_hbm, out_hbm, idxr_v, idxt_v, pos_v, rows_v, out_t, gsem, osem):
    cidx = lax.axis_index("c")
    sidx = lax.axis_index("s")
    w = sidx * 2 + cidx
    b0 = pl.multiple_of(w * B_BLK, 8)

    pltpu.sync_copy(idx_hbm.at[pl.ds(b0, B_BLK)], idxr_v)
    pltpu.sync_copy(pos_hbm, pos_v)

    iota = jnp.arange(16, dtype=jnp.int32)
    eb_lo = iota // 8
    e8_lo = iota % 8
    eb_hi = (iota + 16) // 8
    e8_hi = (iota + 16) % 8

    def tr_body(s, carry):
        svec = jnp.full((16,), 0, jnp.int32) + s
        for g in range(B_BLK // 16):
            v = plsc.load_gather(idxr_v, [iota + 16 * g, svec])
            idxt_v[s, pl.ds(16 * g, 16)] = v
        return carry

    lax.fori_loop(0, SEQ, tr_body, 0)

    def gather_start(s, slot):
        pltpu.async_copy(table_hbm.at[idxt_v.at[s]], rows_v.at[slot], gsem.at[slot])

    def gather_drain(slot):
        pltpu.make_async_copy(
            table_hbm.at[pl.ds(0, B_BLK)], rows_v.at[slot], gsem.at[slot]
        ).wait()

    def out_group_start(g, par):
        pltpu.async_copy(
            out_t.at[par, :, :, :, pl.ds(0, 128)],
            out_hbm.at[pl.ds(NBUF * g, NBUF), :, w],
            osem.at[par],
        )

    def out_group_drain(par):
        pltpu.make_async_copy(
            out_t.at[par, :, :, :, pl.ds(0, 128)],
            out_hbm.at[pl.ds(0, NBUF), :, 0],
            osem.at[par],
        ).wait()

    for b in range(NBUF):
        gather_start(b, b)

    def group(g, carry):
        par = lax.rem(g, 2)

        for b in range(NBUF):
            s = NBUF * g + b
            gather_drain(b)
            p0 = pos_v[s, pl.ds(0, 16)]
            p1 = pos_v[s, pl.ds(16, 16)]
            tile = out_t.at[par, b]
            @plsc.parallel_loop(0, B_BLK, step=1, unroll=8)
            def _(bl):
                blv = jnp.full((16,), 0, jnp.int32) + bl
                v0 = rows_v[b, bl, pl.ds(0, 16)]
                v1 = rows_v[b, bl, pl.ds(16, 16)]
                plsc.store_scatter(tile, [eb_lo, e8_lo, blv], v0 * SCALE + p0)
                plsc.store_scatter(tile, [eb_hi, e8_hi, blv], v1 * SCALE + p1)
            @pl.when(g < NGRP - 1)
            def _():
                gather_start(s + NBUF, b)

        out_group_start(g, par)
        out_group_drain(par)
        return carry

    lax.fori_loop(0, NGRP, group, 0)


@jax.jit
def kernel(inputs, token_table, pos_table):
    tt = token_table
    mesh = plsc.VectorSubcoreMesh(core_axis_name="c", subcore_axis_name="s")
    out5 = pl.kernel(
        _body,
        out_type=jax.ShapeDtypeStruct((SEQ, NEB, BATCH // 128, 8, 128), jnp.float32),
        mesh=mesh,
        scratch_types=[
            pltpu.VMEM((B_BLK, SEQ), jnp.int32),
            pltpu.VMEM((SEQ, B_BLK), jnp.int32),
            pltpu.VMEM((SEQ, DIM), jnp.float32),
            pltpu.VMEM((NBUF, B_BLK, DIM), jnp.float32),
            pltpu.VMEM((2, NBUF, NEB, 8, 129), jnp.float32),
            pltpu.SemaphoreType.DMA((NBUF,)),
            pltpu.SemaphoreType.DMA((2,)),
        ],
        compiler_params=pltpu.CompilerParams(
            use_tc_tiling_on_sc=False, needs_layout_passes=False
        ),
    )(inputs, tt, pos_table)
    return out5.transpose(2, 4, 0, 1, 3).reshape(BATCH, SEQ, DIM)


# speedup vs baseline: 1.3149x; 1.0105x over previous
import jax
import jax.numpy as jnp
from jax import lax
from jax.experimental import pallas as pl
from jax.experimental.layout import Layout, with_layout_constraint
from jax.experimental.pallas import tpu as pltpu
from jax.experimental.pallas import tpu_sc as plsc

SEQ = 200
DIM = 32
BATCH = 4096
VOCAB = 1000000
NW = 32
B_BLK = BATCH // NW
NEB = DIM // 8
SCALE = float(DIM) ** 0.5


NBUF = 4
NGRP = SEQ // NBUF


def _body(idx_hbm, table_hbm, pos_hbm, out_hbm, idxr_v, idxt_v, pos_v, rows_v, out_t, gsem, osem):
    cidx = lax.axis_index("c")
    sidx = lax.axis_index("s")
    w = sidx * 2 + cidx
    b0 = pl.multiple_of(w * B_BLK, 8)

    pltpu.sync_copy(idx_hbm.at[pl.ds(b0, B_BLK)], idxr_v)
    pltpu.sync_copy(pos_hbm, pos_v)

    iota = jnp.arange(16, dtype=jnp.int32)
    eb_lo = iota // 8
    e8_lo = iota % 8
    eb_hi = (iota + 16) // 8
    e8_hi = (iota + 16) % 8

    def tr_body(s, carry):
        svec = jnp.full((16,), 0, jnp.int32) + s
        for g in range(B_BLK // 16):
            v = plsc.load_gather(idxr_v, [iota + 16 * g, svec])
            idxt_v[s, pl.ds(16 * g, 16)] = v
        return carry

    lax.fori_loop(0, SEQ, tr_body, 0)

    def gather_start(s, slot):
        pltpu.async_copy(table_hbm.at[idxt_v.at[s]], rows_v.at[slot], gsem.at[slot])

    def gather_drain(slot):
        pltpu.make_async_copy(
            table_hbm.at[pl.ds(0, B_BLK)], rows_v.at[slot], gsem.at[slot]
        ).wait()

    def out_group_start(g, par):
        pltpu.async_copy(
            out_t.at[par, :, :, :, pl.ds(0, 128)],
            out_hbm.at[pl.ds(NBUF * g, NBUF), :, w],
            osem.at[par],
        )

    def out_group_drain(par):
        pltpu.make_async_copy(
            out_t.at[par, :, :, :, pl.ds(0, 128)],
            out_hbm.at[pl.ds(0, NBUF), :, 0],
            osem.at[par],
        ).wait()

    for b in range(NBUF):
        gather_start(b, b)

    out_group_start(0, 0)
    out_group_start(1, 1)

    def group(g, carry):
        par = lax.rem(g, 2)
        out_group_drain(par)

        for b in range(NBUF):
            s = NBUF * g + b
            gather_drain(b)
            p0 = pos_v[s, pl.ds(0, 16)]
            p1 = pos_v[s, pl.ds(16, 16)]
            tile = out_t.at[par, b]
            @plsc.parallel_loop(0, B_BLK, step=1, unroll=8)
            def _(bl):
                blv = jnp.full((16,), 0, jnp.int32) + bl
                v0 = rows_v[b, bl, pl.ds(0, 16)]
                v1 = rows_v[b, bl, pl.ds(16, 16)]
                plsc.store_scatter(tile, [eb_lo, e8_lo, blv], v0 * SCALE + p0)
                plsc.store_scatter(tile, [eb_hi, e8_hi, blv], v1 * SCALE + p1)
            @pl.when(g < NGRP - 1)
            def _():
                gather_start(s + NBUF, b)

        out_group_start(g, par)
        return carry

    lax.fori_loop(0, NGRP, group, 0)
    out_group_drain(0)
    out_group_drain(1)


@jax.jit
def kernel(inputs, token_table, pos_table):
    tt = token_table
    mesh = plsc.VectorSubcoreMesh(core_axis_name="c", subcore_axis_name="s")
    out5 = pl.kernel(
        _body,
        out_type=jax.ShapeDtypeStruct((SEQ, NEB, BATCH // 128, 8, 128), jnp.float32),
        mesh=mesh,
        scratch_types=[
            pltpu.VMEM((B_BLK, SEQ), jnp.int32),
            pltpu.VMEM((SEQ, B_BLK), jnp.int32),
            pltpu.VMEM((SEQ, DIM), jnp.float32),
            pltpu.VMEM((NBUF, B_BLK, DIM), jnp.float32),
            pltpu.VMEM((2, NBUF, NEB, 8, 129), jnp.float32),
            pltpu.SemaphoreType.DMA((NBUF,)),
            pltpu.SemaphoreType.DMA((2,)),
        ],
        compiler_params=pltpu.CompilerParams(
            use_tc_tiling_on_sc=False, needs_layout_passes=False
        ),
    )(inputs, tt, pos_table)
    return out5.transpose(2, 4, 0, 1, 3).reshape(BATCH, SEQ, DIM)


# speedup vs baseline: 1.3281x; 1.0100x over previous
import jax
import jax.numpy as jnp
from jax import lax
from jax.experimental import pallas as pl
from jax.experimental.layout import Layout, with_layout_constraint
from jax.experimental.pallas import tpu as pltpu
from jax.experimental.pallas import tpu_sc as plsc

SEQ = 200
DIM = 32
BATCH = 4096
VOCAB = 1000000
NW = 32
B_BLK = BATCH // NW
NEB = DIM // 8
SCALE = float(DIM) ** 0.5


NBUF = 4
GDEPTH = 2 * NBUF
NGRP = SEQ // NBUF


def _body(idx_hbm, table_hbm, pos_hbm, out_hbm, idxr_v, idxt_v, pos_v, rows_v, out_t, gsem, osem):
    cidx = lax.axis_index("c")
    sidx = lax.axis_index("s")
    w = sidx * 2 + cidx
    b0 = pl.multiple_of(w * B_BLK, 8)

    pltpu.sync_copy(idx_hbm.at[pl.ds(b0, B_BLK)], idxr_v)
    pltpu.sync_copy(pos_hbm, pos_v)

    iota = jnp.arange(16, dtype=jnp.int32)
    eb_lo = iota // 8
    e8_lo = iota % 8
    eb_hi = (iota + 16) // 8
    e8_hi = (iota + 16) % 8

    def tr_body(s, carry):
        svec = jnp.full((16,), 0, jnp.int32) + s
        for g in range(B_BLK // 16):
            v = plsc.load_gather(idxr_v, [iota + 16 * g, svec])
            idxt_v[s, pl.ds(16 * g, 16)] = v
        return carry

    lax.fori_loop(0, SEQ, tr_body, 0)

    def gather_start(s, slot):
        pltpu.async_copy(table_hbm.at[idxt_v.at[s]], rows_v.at[slot], gsem.at[slot])

    def gather_drain(slot):
        pltpu.make_async_copy(
            table_hbm.at[pl.ds(0, B_BLK)], rows_v.at[slot], gsem.at[slot]
        ).wait()

    def out_group_start(g, par):
        pltpu.async_copy(
            out_t.at[par, :, :, :, pl.ds(0, 128)],
            out_hbm.at[pl.ds(NBUF * g, NBUF), :, w],
            osem.at[par],
        )

    def out_group_drain(par):
        pltpu.make_async_copy(
            out_t.at[par, :, :, :, pl.ds(0, 128)],
            out_hbm.at[pl.ds(0, NBUF), :, 0],
            osem.at[par],
        ).wait()

    for b in range(GDEPTH):
        gather_start(b, b)

    out_group_start(0, 0)
    out_group_start(1, 1)

    def group(g, carry):
        par = lax.rem(g, 2)
        out_group_drain(par)

        for b in range(NBUF):
            s = NBUF * g + b
            slot = NBUF * par + b
            gather_drain(slot)
            p0 = pos_v[s, pl.ds(0, 16)]
            p1 = pos_v[s, pl.ds(16, 16)]
            tile = out_t.at[par, b]
            @plsc.parallel_loop(0, B_BLK, step=1, unroll=8)
            def _(bl):
                blv = jnp.full((16,), 0, jnp.int32) + bl
                v0 = rows_v[slot, bl, pl.ds(0, 16)]
                v1 = rows_v[slot, bl, pl.ds(16, 16)]
                plsc.store_scatter(tile, [eb_lo, e8_lo, blv], v0 * SCALE + p0)
                plsc.store_scatter(tile, [eb_hi, e8_hi, blv], v1 * SCALE + p1)
            @pl.when(g < NGRP - 2)
            def _():
                gather_start(s + GDEPTH, slot)

        out_group_start(g, par)
        return carry

    lax.fori_loop(0, NGRP, group, 0)
    out_group_drain(0)
    out_group_drain(1)


@jax.jit
def kernel(inputs, token_table, pos_table):
    tt = token_table
    mesh = plsc.VectorSubcoreMesh(core_axis_name="c", subcore_axis_name="s")
    out5 = pl.kernel(
        _body,
        out_type=jax.ShapeDtypeStruct((SEQ, NEB, BATCH // 128, 8, 128), jnp.float32),
        mesh=mesh,
        scratch_types=[
            pltpu.VMEM((B_BLK, SEQ), jnp.int32),
            pltpu.VMEM((SEQ, B_BLK), jnp.int32),
            pltpu.VMEM((SEQ, DIM), jnp.float32),
            pltpu.VMEM((GDEPTH, B_BLK, DIM), jnp.float32),
            pltpu.VMEM((2, NBUF, NEB, 8, 129), jnp.float32),
            pltpu.SemaphoreType.DMA((GDEPTH,)),
            pltpu.SemaphoreType.DMA((2,)),
        ],
        compiler_params=pltpu.CompilerParams(
            use_tc_tiling_on_sc=False, needs_layout_passes=False
        ),
    )(inputs, tt, pos_table)
    return out5.transpose(2, 4, 0, 1, 3).reshape(BATCH, SEQ, DIM)
